# Initial kernel scaffold; baseline (speedup 1.0000x reference)
import jax
import jax.numpy as jnp
from jax.experimental import pallas as pl


def kernel(x, edge_index, batch, ne_coeffs, ne_W, ne_b, ne_alpha, ml_coeffs, ml_W, ml_b, ml_alpha, att_W, att_b, r1_W, r1_b, r2_W, r2_b):
    raise NotImplementedError("write your pallas kernel here")


# speedup vs baseline: 12.3325x; 12.3325x over previous
import functools

import jax
import jax.numpy as jnp
from jax import lax
from jax.experimental import pallas as pl
from jax.experimental.pallas import tpu as pltpu
from jax.experimental.pallas import tpu_sc as plsc

N = 10000
E = 320000
H = 128
G = 256
NC = 2
NS = 16
NW = NC * NS
EPW = E // NW
KCH = 128
NCH = 80

BLK = 1000
GRID = N // BLK

_f32 = jnp.float32


def _elu(z):
    return jnp.where(z > 0, z, jnp.exp(z) - 1.0)


def _enc_body(x_ref, a_ref, b_ref, bias_ref, o_ref):
    h = x_ref[...]
    z = (jnp.dot(h, a_ref[...], preferred_element_type=_f32)
         + jnp.dot(h * h, b_ref[...], preferred_element_type=_f32)
         + bias_ref[...])
    o_ref[...] = _elu(z)


def _tc_enc(x, A, B, bias):
    wspec = pl.BlockSpec((H, H), lambda i: (0, 0))
    bspec = pl.BlockSpec((1, H), lambda i: (0, 0))
    rspec = pl.BlockSpec((BLK, H), lambda i: (i, 0))
    return pl.pallas_call(
        _enc_body,
        grid=(GRID,),
        in_specs=[rspec, wspec, wspec, bspec],
        out_specs=rspec,
        out_shape=jax.ShapeDtypeStruct((N, H), _f32),
    )(x, A, B, bias)


def _msg_body(has_agg, *refs):
    if has_agg:
        (h_ref, agga_ref, aggb_ref, a_ref, b_ref, bias_ref, w12_ref,
         bvec_ref, h_out, m_out, sp_out) = refs
        h = h_ref[...] + agga_ref[0] + aggb_ref[0]
        h_out[...] = h
    else:
        h_ref, a_ref, b_ref, bias_ref, w12_ref, bvec_ref, m_out, sp_out = refs
        h = h_ref[...]
    z = (jnp.dot(h, a_ref[...], preferred_element_type=_f32)
         + jnp.dot(h * h, b_ref[...], preferred_element_type=_f32)
         + bias_ref[...])
    m_out[...] = _elu(z)
    sp_out[...] = jnp.dot(h, w12_ref[...], preferred_element_type=_f32) + bvec_ref[...]


def _tc_msg(h, agg2, A, B, bias, w12, bvec):
    wspec = pl.BlockSpec((H, H), lambda i: (0, 0))
    bspec = pl.BlockSpec((1, H), lambda i: (0, 0))
    rspec = pl.BlockSpec((BLK, H), lambda i: (i, 0))
    rout = jax.ShapeDtypeStruct((N, H), _f32)
    if agg2 is None:
        m, sp = pl.pallas_call(
            functools.partial(_msg_body, False),
            grid=(GRID,),
            in_specs=[rspec, wspec, wspec, bspec, wspec, bspec],
            out_specs=[rspec, rspec],
            out_shape=[rout, rout],
        )(h, A, B, bias, w12, bvec)
        return h, m, sp
    aspec = pl.BlockSpec((1, BLK, H), lambda i: (0, i, 0))
    bspec2 = pl.BlockSpec((1, BLK, H), lambda i: (1, i, 0))
    h2, m, sp = pl.pallas_call(
        functools.partial(_msg_body, True),
        grid=(GRID,),
        in_specs=[rspec, aspec, bspec2, wspec, wspec, bspec, wspec, bspec],
        out_specs=[rspec, rspec, rspec],
        out_shape=[rout, rout, rout],
    )(h, agg2, agg2, A, B, bias, w12, bvec)
    return h2, m, sp


def _readout_body(h_ref, agga_ref, aggb_ref, bt_ref, r1_ref, r1b_ref,
                  r2_ref, r2b_ref, o_ref, gs_ref):
    i = pl.program_id(0)
    h3 = h_ref[...] + agga_ref[0] + aggb_ref[0]
    seg = bt_ref[0]
    rows = lax.broadcasted_iota(jnp.int32, (G, BLK), 0)
    onehot = (rows == seg).astype(_f32)
    part = jnp.dot(onehot, h3, preferred_element_type=_f32)

    @pl.when(i == 0)
    def _():
        gs_ref[...] = part

    @pl.when(i > 0)
    def _():
        gs_ref[...] += part

    @pl.when(i == GRID - 1)
    def _():
        g1 = _elu(jnp.dot(gs_ref[...], r1_ref[...],
                          preferred_element_type=_f32) + r1b_ref[...])
        o_ref[...] = jnp.dot(g1, r2_ref[...],
                             preferred_element_type=_f32) + r2b_ref[...]


def _tc_readout(h, agg2, batch_r, r1t, r1b, r2t, r2b):
    wspec = pl.BlockSpec((H, H), lambda i: (0, 0))
    bspec = pl.BlockSpec((1, H), lambda i: (0, 0))
    rspec = pl.BlockSpec((BLK, H), lambda i: (i, 0))
    aspec = pl.BlockSpec((1, BLK, H), lambda i: (0, i, 0))
    aspec2 = pl.BlockSpec((1, BLK, H), lambda i: (1, i, 0))
    btspec = pl.BlockSpec((1, 1, BLK), lambda i: (i, 0, 0))
    ospec = pl.BlockSpec((G, H), lambda i: (0, 0))
    return pl.pallas_call(
        _readout_body,
        grid=(GRID,),
        in_specs=[rspec, aspec, aspec2, btspec, wspec, bspec, wspec, bspec],
        out_specs=ospec,
        out_shape=jax.ShapeDtypeStruct((G, H), _f32),
        scratch_shapes=[pltpu.VMEM((G, H), _f32)],
    )(h, agg2, agg2, batch_r, r1t, r1b, r2t, r2b)


KVAL = 125


def _sc_attn_body(m_hbm, s1_hbm, s2_hbm, isrc_hbm, itgt_hbm, out_hbm,
                  srco_v, tgto_v, alo_v, v1_v, v2_v, zs_v,
                  s1_sh, s2_sh, sums_sh, agg_sh, rows_v, sem):
    c = lax.axis_index("c")
    s = lax.axis_index("s")
    g0 = c * NS + s
    g1 = (1 - c) * NS + s

    zeros16 = jnp.zeros((16,), _f32)
    padmask = (lax.iota(jnp.int32, 16) < KVAL - 7 * 16).astype(_f32)

    @pl.when(s == 0)
    def _():
        pltpu.sync_copy(s1_hbm, s1_sh)
        pltpu.sync_copy(s2_hbm, s2_sh)

    def zloop(i, _):
        zs_v[pl.ds(i * 16, 16)] = zeros16
        return _
    lax.fori_loop(0, 40, zloop, None)

    def zloop2(i, _):
        r = i // 8
        col = (i % 8) * 16
        rows_v[r, pl.ds(col, 16)] = zeros16
        return _
    lax.fori_loop(0, KCH * 8, zloop2, None)

    @pl.when(s < NS - 1)
    def _():
        pltpu.sync_copy(zs_v.at[pl.ds(0, 624)], sums_sh.at[pl.ds(s * 624, 624)])
        for q in range(4):
            pltpu.sync_copy(rows_v,
                            agg_sh.at[pl.ds(s * 624 + q * 128, 128)])
        pltpu.sync_copy(rows_v.at[pl.ds(0, 112)],
                        agg_sh.at[pl.ds(s * 624 + 512, 112)])

    @pl.when(s == NS - 1)
    def _():
        pltpu.sync_copy(zs_v, sums_sh.at[pl.ds(9360, 640)])
        for q in range(5):
            pltpu.sync_copy(rows_v, agg_sh.at[pl.ds(9360 + q * 128, 128)])

    plsc.subcore_barrier()

    def alpha_pass(keep):
        def body(r, _):
            cp1 = pltpu.async_copy(s1_sh.at[srco_v.at[r]], v1_v, sem)
            cp2 = pltpu.async_copy(s2_sh.at[tgto_v.at[r]], v2_v, sem)
            cp1.wait()
            cp2.wait()
            for j in range(KCH // 16):
                sl = pl.ds(j * 16, 16)
                a = v1_v[sl] + v2_v[sl]
                a = jnp.maximum(a, 0.2 * a)
                al = jnp.exp(a)
                if j == KCH // 16 - 1:
                    al = al * padmask
                alo_v[r, sl] = al
            pltpu.sync_copy(alo_v.at[r], sums_sh.at[tgto_v.at[r]], add=True)
            return _
        lax.fori_loop(0, NCH, body, None)

    pltpu.sync_copy(isrc_hbm.at[g1], srco_v)
    pltpu.sync_copy(itgt_hbm.at[g1], tgto_v)
    alpha_pass(False)
    pltpu.sync_copy(isrc_hbm.at[g0], srco_v)
    pltpu.sync_copy(itgt_hbm.at[g0], tgto_v)
    alpha_pass(True)

    plsc.subcore_barrier()

    def p2(r, _):
        pltpu.async_copy(sums_sh.at[tgto_v.at[r]], v1_v, sem).wait()
        for j in range(KCH // 16):
            sl = pl.ds(j * 16, 16)
            alo_v[r, sl] = alo_v[r, sl] / (v1_v[sl] + 1e-16)
        return _
    lax.fori_loop(0, NCH, p2, None)

    lanes0 = jnp.zeros((16,), jnp.int32)

    def p3(r, _):
        pltpu.async_copy(m_hbm.at[srco_v.at[r]], rows_v, sem).wait()
        rsplat = lanes0 + r

        def scale(k, _2):
            w = plsc.load_gather(alo_v, [rsplat, lanes0 + k])
            for v in range(H // 16):
                sl = pl.ds(v * 16, 16)
                rows_v[k, sl] = rows_v[k, sl] * w
            return _2
        lax.fori_loop(0, KCH, scale, None)
        pltpu.sync_copy(rows_v, agg_sh.at[tgto_v.at[r]], add=True)
        return _
    lax.fori_loop(0, NCH, p3, None)

    plsc.subcore_barrier()

    @pl.when(s < NS - 1)
    def _():
        sl = pl.ds(s * 624, 624)
        pltpu.sync_copy(agg_sh.at[sl], out_hbm.at[c].at[sl])

    @pl.when(s == NS - 1)
    def _():
        sl = pl.ds(9360, 640)
        pltpu.sync_copy(agg_sh.at[sl], out_hbm.at[c].at[sl])


def _sc_attn(m, s1, s2, isrc, itgt):
    mesh = plsc.VectorSubcoreMesh(core_axis_name="c", subcore_axis_name="s")
    kern = pl.kernel(
        _sc_attn_body,
        out_type=jax.ShapeDtypeStruct((NC, N, H), _f32),
        mesh=mesh,
        compiler_params=pltpu.CompilerParams(needs_layout_passes=False),
        scratch_types=[
            pltpu.VMEM((NCH, KCH), jnp.int32),
            pltpu.VMEM((NCH, KCH), jnp.int32),
            pltpu.VMEM((NCH, KCH), _f32),
            pltpu.VMEM((KCH,), _f32),
            pltpu.VMEM((KCH,), _f32),
            pltpu.VMEM((640,), _f32),
            pltpu.VMEM_SHARED((N,), _f32),
            pltpu.VMEM_SHARED((N,), _f32),
            pltpu.VMEM_SHARED((N,), _f32),
            pltpu.VMEM_SHARED((N, H), _f32),
            pltpu.VMEM((KCH, H), _f32),
            pltpu.SemaphoreType.DMA,
        ],
    )
    return kern(m, s1, s2, isrc, itgt)


def kernel(x, edge_index, batch, ne_coeffs, ne_W, ne_b, ne_alpha,
           ml_coeffs, ml_W, ml_b, ml_alpha, att_W, att_b,
           r1_W, r1_b, r2_W, r2_b):
    a0 = jax.nn.sigmoid(ne_alpha)
    A0 = (1.0 - a0) * ne_W.T + a0 * ne_coeffs[:, :, 0]
    B0 = a0 * ne_coeffs[:, :, 1]
    bias0 = ((1.0 - a0) * ne_b).reshape(1, H)

    isrc = jnp.pad(edge_index[0].reshape(NW, NCH, KVAL), ((0, 0), (0, 0), (0, 3)))
    itgt = jnp.pad(edge_index[1].reshape(NW, NCH, KVAL), ((0, 0), (0, 0), (0, 3)))
    batch_r = batch.reshape(GRID, 1, BLK)

    r1t = jnp.zeros((H, H), _f32).at[:, : H // 2].set(r1_W.T)
    r1b = jnp.zeros((1, H), _f32).at[0, : H // 2].set(r1_b)
    r2t = jnp.zeros((H, H), _f32).at[: H // 2, 0].set(r2_W[0])
    r2b = jnp.full((1, H), r2_b[0], _f32)

    h = _tc_enc(x, A0, B0, bias0)

    agg2 = None
    for i in range(3):
        ai = jax.nn.sigmoid(ml_alpha[i])
        Ai = (1.0 - ai) * ml_W[i].T + ai * ml_coeffs[i, :, :, 0]
        Bi = ai * ml_coeffs[i, :, :, 1]
        biasi = ((1.0 - ai) * ml_b[i]).reshape(1, H)
        w12 = (jnp.zeros((H, H), _f32)
               .at[:, 0].set(att_W[i, :H])
               .at[:, 1].set(att_W[i, H:]))
        bvec = jnp.zeros((1, H), _f32).at[0, 1].set(att_b[i])
        h, m, sp = _tc_msg(h, agg2, Ai, Bi, biasi, w12, bvec)
        s1 = sp[:, 0]
        s2 = sp[:, 1]
        agg2 = _sc_attn(m, s1, s2, isrc, itgt)

    out2 = _tc_readout(h, agg2, batch_r, r1t, r1b, r2t, r2b)
    return out2[:, 0]


# speedup vs baseline: 12.3364x; 1.0003x over previous
import functools

import jax
import jax.numpy as jnp
from jax import lax
from jax.experimental import pallas as pl
from jax.experimental.pallas import tpu as pltpu
from jax.experimental.pallas import tpu_sc as plsc

N = 10000
E = 320000
H = 128
G = 256
NC = 2
NS = 16
NW = NC * NS
EPW = E // NW
KCH = 128
NCH = 80

BLK = 1000
GRID = N // BLK

_f32 = jnp.float32


def _elu(z):
    return jnp.where(z > 0, z, jnp.exp(z) - 1.0)


def _enc_body(x_ref, a_ref, b_ref, bias_ref, o_ref):
    h = x_ref[...]
    z = (jnp.dot(h, a_ref[...], preferred_element_type=_f32)
         + jnp.dot(h * h, b_ref[...], preferred_element_type=_f32)
         + bias_ref[...])
    o_ref[...] = _elu(z)


def _tc_enc(x, A, B, bias):
    wspec = pl.BlockSpec((H, H), lambda i: (0, 0))
    bspec = pl.BlockSpec((1, H), lambda i: (0, 0))
    rspec = pl.BlockSpec((BLK, H), lambda i: (i, 0))
    return pl.pallas_call(
        _enc_body,
        grid=(GRID,),
        in_specs=[rspec, wspec, wspec, bspec],
        out_specs=rspec,
        out_shape=jax.ShapeDtypeStruct((N, H), _f32),
    )(x, A, B, bias)


def _msg_body(has_agg, *refs):
    if has_agg:
        (h_ref, agga_ref, aggb_ref, a_ref, b_ref, bias_ref, w12_ref,
         bvec_ref, h_out, m_out, sp_out) = refs
        h = h_ref[...] + agga_ref[0] + aggb_ref[0]
        h_out[...] = h
    else:
        h_ref, a_ref, b_ref, bias_ref, w12_ref, bvec_ref, m_out, sp_out = refs
        h = h_ref[...]
    z = (jnp.dot(h, a_ref[...], preferred_element_type=_f32)
         + jnp.dot(h * h, b_ref[...], preferred_element_type=_f32)
         + bias_ref[...])
    m_out[...] = _elu(z)
    sp_out[...] = jnp.dot(h, w12_ref[...], preferred_element_type=_f32) + bvec_ref[...]


def _tc_msg(h, agg2, A, B, bias, w12, bvec):
    wspec = pl.BlockSpec((H, H), lambda i: (0, 0))
    bspec = pl.BlockSpec((1, H), lambda i: (0, 0))
    rspec = pl.BlockSpec((BLK, H), lambda i: (i, 0))
    rout = jax.ShapeDtypeStruct((N, H), _f32)
    if agg2 is None:
        m, sp = pl.pallas_call(
            functools.partial(_msg_body, False),
            grid=(GRID,),
            in_specs=[rspec, wspec, wspec, bspec, wspec, bspec],
            out_specs=[rspec, rspec],
            out_shape=[rout, rout],
        )(h, A, B, bias, w12, bvec)
        return h, m, sp
    aspec = pl.BlockSpec((1, BLK, H), lambda i: (0, i, 0))
    bspec2 = pl.BlockSpec((1, BLK, H), lambda i: (1, i, 0))
    h2, m, sp = pl.pallas_call(
        functools.partial(_msg_body, True),
        grid=(GRID,),
        in_specs=[rspec, aspec, bspec2, wspec, wspec, bspec, wspec, bspec],
        out_specs=[rspec, rspec, rspec],
        out_shape=[rout, rout, rout],
    )(h, agg2, agg2, A, B, bias, w12, bvec)
    return h2, m, sp


def _readout_body(h_ref, agga_ref, aggb_ref, bt_ref, r1_ref, r1b_ref,
                  r2_ref, r2b_ref, o_ref, gs_ref):
    i = pl.program_id(0)
    h3 = h_ref[...] + agga_ref[0] + aggb_ref[0]
    seg = bt_ref[0]
    rows = lax.broadcasted_iota(jnp.int32, (G, BLK), 0)
    onehot = (rows == seg).astype(_f32)
    part = jnp.dot(onehot, h3, preferred_element_type=_f32)

    @pl.when(i == 0)
    def _():
        gs_ref[...] = part

    @pl.when(i > 0)
    def _():
        gs_ref[...] += part

    @pl.when(i == GRID - 1)
    def _():
        g1 = _elu(jnp.dot(gs_ref[...], r1_ref[...],
                          preferred_element_type=_f32) + r1b_ref[...])
        o_ref[...] = jnp.dot(g1, r2_ref[...],
                             preferred_element_type=_f32) + r2b_ref[...]


def _tc_readout(h, agg2, batch_r, r1t, r1b, r2t, r2b):
    wspec = pl.BlockSpec((H, H), lambda i: (0, 0))
    bspec = pl.BlockSpec((1, H), lambda i: (0, 0))
    rspec = pl.BlockSpec((BLK, H), lambda i: (i, 0))
    aspec = pl.BlockSpec((1, BLK, H), lambda i: (0, i, 0))
    aspec2 = pl.BlockSpec((1, BLK, H), lambda i: (1, i, 0))
    btspec = pl.BlockSpec((1, 1, BLK), lambda i: (i, 0, 0))
    ospec = pl.BlockSpec((G, H), lambda i: (0, 0))
    return pl.pallas_call(
        _readout_body,
        grid=(GRID,),
        in_specs=[rspec, aspec, aspec2, btspec, wspec, bspec, wspec, bspec],
        out_specs=ospec,
        out_shape=jax.ShapeDtypeStruct((G, H), _f32),
        scratch_shapes=[pltpu.VMEM((G, H), _f32)],
    )(h, agg2, agg2, batch_r, r1t, r1b, r2t, r2b)


KVAL = 125


def _sc_attn_body(m_hbm, s1_hbm, s2_hbm, isrc_hbm, itgt_hbm, out_hbm,
                  srco_v, tgto_v, alo_v, v1_v, v2_v, zs_v,
                  s1_sh, s2_sh, sums_sh, agg_sh, rowsa_v, rowsb_v,
                  sem, sema, semb):
    c = lax.axis_index("c")
    s = lax.axis_index("s")
    g0 = c * NS + s
    g1 = (1 - c) * NS + s

    zeros16 = jnp.zeros((16,), _f32)
    padmask = (lax.iota(jnp.int32, 16) < KVAL - 7 * 16).astype(_f32)

    @pl.when(s == 0)
    def _():
        pltpu.sync_copy(s1_hbm, s1_sh)
        pltpu.sync_copy(s2_hbm, s2_sh)

    def zloop(i, _):
        zs_v[pl.ds(i * 16, 16)] = zeros16
        return _
    lax.fori_loop(0, 40, zloop, None)

    def zloop2(i, _):
        r = i // 8
        col = (i % 8) * 16
        rowsa_v[r, pl.ds(col, 16)] = zeros16
        return _
    lax.fori_loop(0, 64 * 8, zloop2, None)

    @pl.when(s < NS - 1)
    def _():
        pltpu.sync_copy(zs_v.at[pl.ds(0, 624)], sums_sh.at[pl.ds(s * 624, 624)])
        for q in range(9):
            pltpu.sync_copy(rowsa_v,
                            agg_sh.at[pl.ds(s * 624 + q * 64, 64)])
        pltpu.sync_copy(rowsa_v.at[pl.ds(0, 48)],
                        agg_sh.at[pl.ds(s * 624 + 576, 48)])

    @pl.when(s == NS - 1)
    def _():
        pltpu.sync_copy(zs_v, sums_sh.at[pl.ds(9360, 640)])
        for q in range(10):
            pltpu.sync_copy(rowsa_v, agg_sh.at[pl.ds(9360 + q * 64, 64)])

    plsc.subcore_barrier()

    def alpha_pass(keep):
        def body(r, _):
            cp1 = pltpu.async_copy(s1_sh.at[srco_v.at[r]], v1_v, sem)
            cp2 = pltpu.async_copy(s2_sh.at[tgto_v.at[r]], v2_v, sem)
            cp1.wait()
            cp2.wait()
            for j in range(KCH // 16):
                sl = pl.ds(j * 16, 16)
                a = v1_v[sl] + v2_v[sl]
                a = jnp.maximum(a, 0.2 * a)
                al = jnp.exp(a)
                if j == KCH // 16 - 1:
                    al = al * padmask
                alo_v[r, sl] = al
            pltpu.sync_copy(alo_v.at[r], sums_sh.at[tgto_v.at[r]], add=True)
            return _
        lax.fori_loop(0, NCH, body, None)

    pltpu.sync_copy(isrc_hbm.at[g1], srco_v)
    pltpu.sync_copy(itgt_hbm.at[g1], tgto_v)
    alpha_pass(False)
    pltpu.sync_copy(isrc_hbm.at[g0], srco_v)
    pltpu.sync_copy(itgt_hbm.at[g0], tgto_v)
    alpha_pass(True)

    plsc.subcore_barrier()

    def p2(r, _):
        pltpu.async_copy(sums_sh.at[tgto_v.at[r]], v1_v, sem).wait()
        for j in range(KCH // 16):
            sl = pl.ds(j * 16, 16)
            alo_v[r, sl] = alo_v[r, sl] / (v1_v[sl] + 1e-16)
        return _
    lax.fori_loop(0, NCH, p2, None)

    lanes0 = jnp.zeros((16,), jnp.int32)

    def issue(r, p, buf, bsem):
        idx = srco_v.at[r, pl.ds(p * 64, 64)]
        return pltpu.async_copy(m_hbm.at[idx], buf, bsem)

    def drain(buf, bsem):
        pltpu.make_async_copy(m_hbm.at[pl.ds(0, 64)], buf, bsem).wait()

    def scale_scatter(r, p, buf):
        rsplat = lanes0 + r

        def scale(k, _2):
            w = plsc.load_gather(alo_v, [rsplat, lanes0 + (p * 64 + k)])
            for v in range(H // 16):
                sl = pl.ds(v * 16, 16)
                buf[k, sl] = buf[k, sl] * w
            return _2
        lax.fori_loop(0, 64, scale, None)
        pltpu.sync_copy(buf, agg_sh.at[tgto_v.at[r, pl.ds(p * 64, 64)]],
                        add=True)

    issue(0, 0, rowsa_v, sema)

    def p3(r, _):
        drain(rowsa_v, sema)
        issue(r, 1, rowsb_v, semb)
        scale_scatter(r, 0, rowsa_v)
        drain(rowsb_v, semb)

        @pl.when(r + 1 < NCH)
        def _():
            issue(r + 1, 0, rowsa_v, sema)
        scale_scatter(r, 1, rowsb_v)
        return _
    lax.fori_loop(0, NCH, p3, None)

    plsc.subcore_barrier()

    @pl.when(s < NS - 1)
    def _():
        sl = pl.ds(s * 624, 624)
        pltpu.sync_copy(agg_sh.at[sl], out_hbm.at[c].at[sl])

    @pl.when(s == NS - 1)
    def _():
        sl = pl.ds(9360, 640)
        pltpu.sync_copy(agg_sh.at[sl], out_hbm.at[c].at[sl])


def _sc_attn(m, s1, s2, isrc, itgt):
    mesh = plsc.VectorSubcoreMesh(core_axis_name="c", subcore_axis_name="s")
    kern = pl.kernel(
        _sc_attn_body,
        out_type=jax.ShapeDtypeStruct((NC, N, H), _f32),
        mesh=mesh,
        compiler_params=pltpu.CompilerParams(needs_layout_passes=False),
        scratch_types=[
            pltpu.VMEM((NCH, KCH), jnp.int32),
            pltpu.VMEM((NCH, KCH), jnp.int32),
            pltpu.VMEM((NCH, KCH), _f32),
            pltpu.VMEM((KCH,), _f32),
            pltpu.VMEM((KCH,), _f32),
            pltpu.VMEM((640,), _f32),
            pltpu.VMEM_SHARED((N,), _f32),
            pltpu.VMEM_SHARED((N,), _f32),
            pltpu.VMEM_SHARED((N,), _f32),
            pltpu.VMEM_SHARED((N, H), _f32),
            pltpu.VMEM((64, H), _f32),
            pltpu.VMEM((64, H), _f32),
            pltpu.SemaphoreType.DMA,
            pltpu.SemaphoreType.DMA,
            pltpu.SemaphoreType.DMA,
        ],
    )
    return kern(m, s1, s2, isrc, itgt)


def kernel(x, edge_index, batch, ne_coeffs, ne_W, ne_b, ne_alpha,
           ml_coeffs, ml_W, ml_b, ml_alpha, att_W, att_b,
           r1_W, r1_b, r2_W, r2_b):
    a0 = jax.nn.sigmoid(ne_alpha)
    A0 = (1.0 - a0) * ne_W.T + a0 * ne_coeffs[:, :, 0]
    B0 = a0 * ne_coeffs[:, :, 1]
    bias0 = ((1.0 - a0) * ne_b).reshape(1, H)

    isrc = jnp.pad(edge_index[0].reshape(NW, NCH, KVAL),
                   ((0, 0), (0, 0), (0, 3)))
    itgt = jnp.pad(edge_index[1].reshape(NW, NCH, KVAL),
                   ((0, 0), (0, 0), (0, 3)))
    batch_r = batch.reshape(GRID, 1, BLK)

    r1t = jnp.zeros((H, H), _f32).at[:, : H // 2].set(r1_W.T)
    r1b = jnp.zeros((1, H), _f32).at[0, : H // 2].set(r1_b)
    r2t = jnp.zeros((H, H), _f32).at[: H // 2, 0].set(r2_W[0])
    r2b = jnp.full((1, H), r2_b[0], _f32)

    h = _tc_enc(x, A0, B0, bias0)

    agg2 = None
    for i in range(3):
        ai = jax.nn.sigmoid(ml_alpha[i])
        Ai = (1.0 - ai) * ml_W[i].T + ai * ml_coeffs[i, :, :, 0]
        Bi = ai * ml_coeffs[i, :, :, 1]
        biasi = ((1.0 - ai) * ml_b[i]).reshape(1, H)
        w12 = (jnp.zeros((H, H), _f32)
               .at[:, 0].set(att_W[i, :H])
               .at[:, 1].set(att_W[i, H:]))
        bvec = jnp.zeros((1, H), _f32).at[0, 1].set(att_b[i])
        h, m, sp = _tc_msg(h, agg2, Ai, Bi, biasi, w12, bvec)
        s1 = sp[:, 0]
        s2 = sp[:, 1]
        agg2 = _sc_attn(m, s1, s2, isrc, itgt)

    out2 = _tc_readout(h, agg2, batch_r, r1t, r1b, r2t, r2b)
    return out2[:, 0]
